# Initial kernel scaffold; baseline (speedup 1.0000x reference)
import jax
import jax.numpy as jnp
from jax.experimental import pallas as pl


def kernel(input_ids, token_type_ids, token_table, seg_table, pe):
    raise NotImplementedError("write your pallas kernel here")


# speedup vs baseline: 3.2419x; 3.2419x over previous
import functools

import jax
import jax.numpy as jnp
from jax import lax
from jax.experimental import pallas as pl
from jax.experimental.pallas import tpu as pltpu
from jax.experimental.pallas import tpu_sc as plsc

_NC, _NS, _LANES = 2, 16, 16
_NW = _NC * _NS


def _prep_body(tt_ref, seg2_ref, pe_ref, combo_ref, cidx_ref):
    seq_len = pe_ref.shape[0]
    pe = pe_ref[...]
    combo_ref[0:seq_len, :] = pe + seg2_ref[0:1, :]
    combo_ref[seq_len : 2 * seq_len, :] = pe + seg2_ref[1:2, :]
    cidx_ref[...] = tt_ref[...] * seq_len + lax.broadcasted_iota(
        jnp.int32, tt_ref.shape, 1
    )


def _make_sc_gather(n_rows, d_model, n_combo, chunk):
    rpw = n_rows // _NW
    nch = rpw // chunk
    assert rpw % chunk == 0 and chunk % 8 == 0 and nch % 2 == 0

    mesh = plsc.VectorSubcoreMesh(core_axis_name="c", subcore_axis_name="s")

    @functools.partial(
        pl.kernel,
        out_type=jax.ShapeDtypeStruct((n_rows, d_model), jnp.float32),
        mesh=mesh,
        scratch_types=[
            pltpu.VMEM((rpw,), jnp.int32),
            pltpu.VMEM((rpw + _LANES,), jnp.int32),
            pltpu.VMEM((n_combo, d_model), jnp.float32),
            pltpu.VMEM((2, chunk, d_model), jnp.float32),
            pltpu.SemaphoreType.DMA,
            pltpu.SemaphoreType.DMA,
            pltpu.SemaphoreType.DMA,
            pltpu.SemaphoreType.DMA,
        ],
    )
    def sc_gather(ids_hbm, cidx_hbm, token_hbm, combo_hbm, out_hbm,
                  idx_v, cidx_v, combo_v, buf,
                  sem_t0, sem_t1, sem_o0, sem_o1):
        wid = lax.axis_index("s") * _NC + lax.axis_index("c")
        base = wid * rpw
        pltpu.sync_copy(combo_hbm, combo_v)
        pltpu.sync_copy(ids_hbm.at[pl.ds(base, rpw)], idx_v)
        pltpu.sync_copy(cidx_hbm.at[pl.ds(base, rpw)], cidx_v.at[pl.ds(0, rpw)])

        sem_t = (sem_t0, sem_t1)
        sem_o = (sem_o0, sem_o1)

        def start_gather(g, s):
            return pltpu.async_copy(
                token_hbm.at[idx_v.at[pl.ds(g * chunk, chunk)]],
                buf.at[s], sem_t[s])

        def add_combo(g, s):
            def row_body(r, _):
                cvec = cidx_v[pl.ds(g * chunk + r, _LANES)]
                c_r = cvec[0]
                for j in range(d_model // _LANES):
                    sl = pl.ds(j * _LANES, _LANES)
                    buf[s, r, sl] = buf[s, r, sl] + combo_v[c_r, sl]
                return 0

            lax.fori_loop(0, chunk, row_body, 0)

        def wait_gather(g, s):
            pltpu.make_async_copy(
                token_hbm.at[idx_v.at[pl.ds(g * chunk, chunk)]],
                buf.at[s], sem_t[s]).wait()

        def wait_out(g, s):
            pltpu.make_async_copy(
                buf.at[s], out_hbm.at[pl.ds(base + g * chunk, chunk)],
                sem_o[s]).wait()

        def issue_out(g, s):
            pltpu.async_copy(
                buf.at[s], out_hbm.at[pl.ds(base + g * chunk, chunk)],
                sem_o[s])

        start_gather(0, 0)

        def pair_body(p, _):
            g0 = p * 2
            @pl.when(p >= 1)
            def _():
                wait_out(g0 - 1, 1)
            start_gather(g0 + 1, 1)
            wait_gather(g0, 0)
            add_combo(g0, 0)
            issue_out(g0, 0)
            @pl.when(p + 1 < nch // 2)
            def _():
                wait_out(g0, 0)
                start_gather(g0 + 2, 0)
            wait_gather(g0 + 1, 1)
            add_combo(g0 + 1, 1)
            issue_out(g0 + 1, 1)
            return 0

        lax.fori_loop(0, nch // 2, pair_body, 0)
        wait_out(nch - 2, 0)
        wait_out(nch - 1, 1)

    return sc_gather


def kernel(input_ids, token_type_ids, token_table, seg_table, pe):
    b, seq_len = input_ids.shape
    d_model = token_table.shape[1]
    n_rows = b * seq_len

    tt = token_type_ids.astype(jnp.int32)
    seg2 = seg_table[:2]
    pe_l = pe[:seq_len]

    combo, cidx = pl.pallas_call(
        _prep_body,
        out_shape=[
            jax.ShapeDtypeStruct((2 * seq_len, d_model), jnp.float32),
            jax.ShapeDtypeStruct((b, seq_len), jnp.int32),
        ],
    )(tt, seg2, pe_l)

    ids_flat = input_ids.reshape(n_rows).astype(jnp.int32)
    cidx_flat = cidx.reshape(n_rows)

    sc_gather = _make_sc_gather(n_rows, d_model, 2 * seq_len, chunk=40)
    out = sc_gather(ids_flat, cidx_flat, token_table, combo)
    return out.reshape(b, seq_len, d_model)


# speedup vs baseline: 7.2006x; 2.2211x over previous
import functools

import jax
import jax.numpy as jnp
from jax import lax
from jax.experimental import pallas as pl
from jax.experimental.pallas import tpu as pltpu
from jax.experimental.pallas import tpu_sc as plsc

_NC, _NS, _LANES = 2, 16, 16
_NW = _NC * _NS


def _prep_body(tt_ref, seg2_ref, pe_ref, combo_ref, cidx_ref, oidx_ref):
    seq_len = pe_ref.shape[0]
    batch = tt_ref.shape[0]
    pe = pe_ref[...]
    combo_ref[0:seq_len, :] = pe + seg2_ref[0:1, :]
    combo_ref[seq_len : 2 * seq_len, :] = pe + seg2_ref[1:2, :]
    l_iota = lax.broadcasted_iota(jnp.int32, (batch, seq_len), 1)
    b_iota = lax.broadcasted_iota(jnp.int32, (batch, seq_len), 0)
    cidx_ref[...] = tt_ref[...] * seq_len + l_iota
    oidx_ref[...] = l_iota * batch + b_iota


def _make_sc_gather(n_rows, d_model, chunk):
    rpw = n_rows // _NW
    nch = rpw // chunk
    assert rpw % chunk == 0 and chunk % 8 == 0 and chunk <= 128

    mesh = plsc.VectorSubcoreMesh(core_axis_name="c", subcore_axis_name="s")

    @functools.partial(
        pl.kernel,
        out_type=jax.ShapeDtypeStruct((n_rows, d_model), jnp.float32),
        mesh=mesh,
        scratch_types=[
            pltpu.VMEM((rpw,), jnp.int32),
            pltpu.VMEM((rpw,), jnp.int32),
            pltpu.VMEM((nch, chunk), jnp.int32),
            pltpu.VMEM((chunk, d_model), jnp.float32),
            pltpu.VMEM((chunk, d_model), jnp.float32),
            pltpu.SemaphoreType.DMA,
            pltpu.SemaphoreType.DMA,
            pltpu.SemaphoreType.DMA,
        ],
    )
    def sc_gather(ids_hbm, cidx_hbm, oidx_hbm, token_hbm, combo_hbm, out_hbm,
                  idx_v, cidx_v, oidx_v, tok_buf, add_buf, sem1, sem2, sem3):
        wid = lax.axis_index("s") * _NC + lax.axis_index("c")
        base = wid * rpw
        pltpu.sync_copy(ids_hbm.at[pl.ds(base, rpw)], idx_v)
        pltpu.sync_copy(cidx_hbm.at[pl.ds(base, rpw)], cidx_v)
        pltpu.sync_copy(oidx_hbm.at[wid], oidx_v)

        def chunk_body(g, _):
            cp1 = pltpu.async_copy(
                token_hbm.at[idx_v.at[pl.ds(g * chunk, chunk)]], tok_buf, sem1)
            cp2 = pltpu.async_copy(
                combo_hbm.at[cidx_v.at[pl.ds(g * chunk, chunk)]], add_buf, sem2)
            cp1.wait()
            cp2.wait()

            def row_body(r, _):
                for j in range(d_model // _LANES):
                    sl = pl.ds(j * _LANES, _LANES)
                    tok_buf[r, sl] = tok_buf[r, sl] + add_buf[r, sl]
                return 0

            lax.fori_loop(0, chunk, row_body, 0)
            pltpu.async_copy(tok_buf, out_hbm.at[oidx_v.at[g]], sem3).wait()
            return 0

        lax.fori_loop(0, nch, chunk_body, 0)

    return sc_gather


def kernel(input_ids, token_type_ids, token_table, seg_table, pe):
    b, seq_len = input_ids.shape
    d_model = token_table.shape[1]
    n_rows = b * seq_len
    chunk = 64

    tt = token_type_ids.astype(jnp.int32)
    seg2 = seg_table[:2]
    pe_l = pe[:seq_len]

    combo, cidx, oidx = pl.pallas_call(
        _prep_body,
        out_shape=[
            jax.ShapeDtypeStruct((2 * seq_len, d_model), jnp.float32),
            jax.ShapeDtypeStruct((b, seq_len), jnp.int32),
            jax.ShapeDtypeStruct((b, seq_len), jnp.int32),
        ],
    )(tt, seg2, pe_l)

    ids_flat = input_ids.reshape(n_rows).astype(jnp.int32)
    cidx_flat = cidx.reshape(n_rows)
    rpw = n_rows // _NW
    oidx_3d = oidx.reshape(_NW, rpw // chunk, chunk)

    sc_gather = _make_sc_gather(n_rows, d_model, chunk)
    out = sc_gather(ids_flat, cidx_flat, oidx_3d, token_table, combo)
    return out.reshape(seq_len, b, d_model).transpose(1, 0, 2)


# speedup vs baseline: 7.4716x; 1.0376x over previous
import functools

import jax
import jax.numpy as jnp
from jax import lax
from jax.experimental import pallas as pl
from jax.experimental.pallas import tpu as pltpu
from jax.experimental.pallas import tpu_sc as plsc

_NC, _NS, _LANES = 2, 16, 16
_NW = _NC * _NS


def _prep_body(tt_ref, seg2_ref, pe_ref, combo_ref, cidx_ref, oidx_ref):
    seq_len = pe_ref.shape[0]
    batch = tt_ref.shape[0]
    pe = pe_ref[...]
    combo_ref[0:seq_len, :] = pe + seg2_ref[0:1, :]
    combo_ref[seq_len : 2 * seq_len, :] = pe + seg2_ref[1:2, :]
    l_iota = lax.broadcasted_iota(jnp.int32, (batch, seq_len), 1)
    b_iota = lax.broadcasted_iota(jnp.int32, (batch, seq_len), 0)
    cidx_ref[...] = tt_ref[...] * seq_len + l_iota
    oidx_ref[...] = l_iota * batch + b_iota


def _make_sc_gather(n_rows, d_model, chunk):
    rpw = n_rows // _NW
    nch = rpw // chunk
    assert rpw % chunk == 0 and chunk % 8 == 0 and chunk <= 128 and nch % 2 == 0

    mesh = plsc.VectorSubcoreMesh(core_axis_name="c", subcore_axis_name="s")

    @functools.partial(
        pl.kernel,
        out_type=jax.ShapeDtypeStruct((n_rows, d_model), jnp.float32),
        mesh=mesh,
        scratch_types=[
            pltpu.VMEM((rpw,), jnp.int32),
            pltpu.VMEM((rpw,), jnp.int32),
            pltpu.VMEM((nch, chunk), jnp.int32),
            pltpu.VMEM((2, chunk, d_model), jnp.float32),
            pltpu.VMEM((2, chunk, d_model), jnp.float32),
            pltpu.SemaphoreType.DMA,
            pltpu.SemaphoreType.DMA,
            pltpu.SemaphoreType.DMA,
            pltpu.SemaphoreType.DMA,
            pltpu.SemaphoreType.DMA,
            pltpu.SemaphoreType.DMA,
        ],
    )
    def sc_gather(ids_hbm, cidx_hbm, oidx_hbm, token_hbm, combo_hbm, out_hbm,
                  idx_v, cidx_v, oidx_v, tok_buf, add_buf,
                  sem_t0, sem_t1, sem_c0, sem_c1, sem_o0, sem_o1):
        wid = lax.axis_index("s") * _NC + lax.axis_index("c")
        base = wid * rpw
        pltpu.sync_copy(ids_hbm.at[pl.ds(base, rpw)], idx_v)
        pltpu.sync_copy(cidx_hbm.at[pl.ds(base, rpw)], cidx_v)
        pltpu.sync_copy(oidx_hbm.at[wid], oidx_v)

        sem_t = (sem_t0, sem_t1)
        sem_c = (sem_c0, sem_c1)
        sem_o = (sem_o0, sem_o1)

        def issue_gathers(g, s):
            pltpu.async_copy(
                token_hbm.at[idx_v.at[pl.ds(g * chunk, chunk)]],
                tok_buf.at[s], sem_t[s])
            pltpu.async_copy(
                combo_hbm.at[cidx_v.at[pl.ds(g * chunk, chunk)]],
                add_buf.at[s], sem_c[s])

        def wait_gathers(g, s):
            pltpu.make_async_copy(
                token_hbm.at[idx_v.at[pl.ds(g * chunk, chunk)]],
                tok_buf.at[s], sem_t[s]).wait()
            pltpu.make_async_copy(
                combo_hbm.at[cidx_v.at[pl.ds(g * chunk, chunk)]],
                add_buf.at[s], sem_c[s]).wait()

        def issue_scatter(g, s):
            pltpu.async_copy(tok_buf.at[s], out_hbm.at[oidx_v.at[g]], sem_o[s])

        def wait_scatter(g, s):
            pltpu.make_async_copy(
                tok_buf.at[s], out_hbm.at[oidx_v.at[g]], sem_o[s]).wait()

        def add_rows(s):
            def row_body(r, _):
                for j in range(d_model // _LANES):
                    sl = pl.ds(j * _LANES, _LANES)
                    plsc.addupdate(tok_buf.at[s, r, sl], add_buf[s, r, sl])
                return 0

            lax.fori_loop(0, chunk, row_body, 0)

        npair = nch // 2
        issue_gathers(0, 0)

        def pair_body(p, _):
            g0 = p * 2

            @pl.when(p >= 1)
            def _():
                wait_scatter(g0 - 1, 1)
            issue_gathers(g0 + 1, 1)
            wait_gathers(g0, 0)
            add_rows(0)
            issue_scatter(g0, 0)

            @pl.when(p + 1 < npair)
            def _():
                wait_scatter(g0, 0)
                issue_gathers(g0 + 2, 0)
            wait_gathers(g0 + 1, 1)
            add_rows(1)
            issue_scatter(g0 + 1, 1)
            return 0

        lax.fori_loop(0, npair, pair_body, 0)
        wait_scatter(nch - 2, 0)
        wait_scatter(nch - 1, 1)

    return sc_gather


def kernel(input_ids, token_type_ids, token_table, seg_table, pe):
    b, seq_len = input_ids.shape
    d_model = token_table.shape[1]
    n_rows = b * seq_len
    chunk = 40

    tt = token_type_ids.astype(jnp.int32)
    seg2 = seg_table[:2]
    pe_l = pe[:seq_len]

    combo, cidx, oidx = pl.pallas_call(
        _prep_body,
        out_shape=[
            jax.ShapeDtypeStruct((2 * seq_len, d_model), jnp.float32),
            jax.ShapeDtypeStruct((b, seq_len), jnp.int32),
            jax.ShapeDtypeStruct((b, seq_len), jnp.int32),
        ],
    )(tt, seg2, pe_l)

    ids_flat = input_ids.reshape(n_rows).astype(jnp.int32)
    cidx_flat = cidx.reshape(n_rows)
    rpw = n_rows // _NW
    oidx_3d = oidx.reshape(_NW, rpw // chunk, chunk)

    sc_gather = _make_sc_gather(n_rows, d_model, chunk)
    out = sc_gather(ids_flat, cidx_flat, oidx_3d, token_table, combo)
    return out.reshape(seq_len, b, d_model).transpose(1, 0, 2)
